# baseline (speedup 1.0000x reference)
import jax
import jax.numpy as jnp
from jax import lax
from jax.experimental import pallas as pl
from jax.experimental.pallas import tpu as pltpu
from jax.experimental.pallas import tpu_sc as plsc

IN_DIM = 65536
OUT_DIM = 16384
COLS = 1024

NC = 2
NS = 16
L = 16

CW = 32
NPASS = COLS // (CW * NC)
RPT = IN_DIM // NS
NBLK = RPT // 128
CHUNK = 512
ZROWS = 128
OBLK = 128


def _body(x_hbm, s_hbm, idx_hbm, z_hbm, y_hbm,
          idx2_v, s_v, xbuf0, xbuf1, obuf0, nbuf0, obuf1, nbuf1, acc,
          sem_z, sem_x0, sem_x1, sem_sc0, sem_sc1, sem_d0, sem_d1, sem_y):
    c = lax.axis_index("c")
    sid = lax.axis_index("s")
    rowbase = sid * RPT

    for b in range(NBLK):
        pltpu.sync_copy(idx_hbm.at[pl.ds(rowbase + b * 128, 128)],
                        idx2_v.at[b])
        pltpu.sync_copy(s_hbm.at[pl.ds(rowbase + b * 128, 128)],
                        s_v.at[b])

    for b in range(NBLK):
        for j in range(128 // L):
            iv = idx2_v[b, pl.ds(j * L, L)]
            sv = s_v[b, pl.ds(j * L, L)]
            bump = jnp.where(sv < 0.0, jnp.int32(OUT_DIM), jnp.int32(0))
            idx2_v[b, pl.ds(j * L, L)] = iv + bump

    zdescs = [
        pltpu.async_copy(
            z_hbm.at[pl.ds(sid * (2 * OUT_DIM // NS) + z * ZROWS, ZROWS)],
            acc.at[pl.ds(sid * (2 * OUT_DIM // NS) + z * ZROWS, ZROWS)],
            sem_z)
        for z in range(2 * OUT_DIM // NS // ZROWS)
    ]
    for dsc in zdescs:
        dsc.wait()
    plsc.subcore_barrier()

    xb = (xbuf0, xbuf1)
    sx = (sem_x0, sem_x1)
    ssc = (sem_sc0, sem_sc1)
    NCH = RPT // CHUNK
    SPC = CHUNK // 128

    def stage(coloff, b2):
        par = b2 % 2
        return pltpu.async_copy(
            x_hbm.at[pl.ds(rowbase + b2 * CHUNK, CHUNK),
                     pl.ds(coloff, CW)],
            xb[par], sx[par])

    def scatter_phase(coloff, pre0, pre1):
        xdescs = {0: pre0, 1: pre1}
        scat_descs = {0: [], 1: []}
        for b2 in range(NCH):
            par = b2 % 2
            xdescs.pop(b2).wait()
            if 1 <= b2 and b2 + 1 < NCH:
                npar = (b2 + 1) % 2
                for dsc in scat_descs[npar]:
                    dsc.wait()
                scat_descs[npar] = []
                xdescs[b2 + 1] = stage(coloff, b2 + 1)
            for b3 in range(SPC):
                scat_descs[par].append(pltpu.async_copy(
                    xb[par].at[pl.ds(b3 * 128, 128)],
                    acc.at[idx2_v.at[b2 * SPC + b3]],
                    ssc[par], add=True))
        for par in (0, 1):
            for dsc in scat_descs[par]:
                dsc.wait()

    def drain_phase(coloff):
        orows = OUT_DIM // NS
        bufs = ((obuf0, nbuf0), (obuf1, nbuf1))
        NOB = orows // OBLK
        rdescs = {}
        ydescs = {}
        zdescs = []

        def issue_read(ob):
            o_, n_ = bufs[ob % 2]
            obase = sid * orows + ob * OBLK
            rdescs[ob] = (
                pltpu.async_copy(acc.at[pl.ds(obase, OBLK)], o_, sem_d0),
                pltpu.async_copy(acc.at[pl.ds(OUT_DIM + obase, OBLK)],
                                 n_, sem_d1))

        issue_read(0)
        for ob in range(NOB):
            o_, n_ = bufs[ob % 2]
            for dsc in rdescs.pop(ob):
                dsc.wait()
            obase = sid * orows + ob * OBLK
            zdescs.append(pltpu.async_copy(
                z_hbm.at[pl.ds(obase, OBLK)],
                acc.at[pl.ds(obase, OBLK)], sem_z))
            zdescs.append(pltpu.async_copy(
                z_hbm.at[pl.ds(OUT_DIM + obase, OBLK)],
                acc.at[pl.ds(OUT_DIM + obase, OBLK)], sem_z))
            if ob + 1 < NOB:
                if ob - 1 >= 0:
                    ydescs.pop(ob - 1).wait()
                issue_read(ob + 1)

            def sub_row(r, _):
                for j in range(CW // L):
                    o_[r, pl.ds(j * L, L)] = (o_[r, pl.ds(j * L, L)]
                                              - n_[r, pl.ds(j * L, L)])
                return _
            lax.fori_loop(0, OBLK, sub_row, None)

            ydescs[ob] = pltpu.async_copy(
                o_,
                y_hbm.at[pl.ds(sid * orows + ob * OBLK, OBLK),
                         pl.ds(coloff, CW)],
                sem_y)
        for dsc in ydescs.values():
            dsc.wait()
        for dsc in zdescs:
            dsc.wait()

    col0 = c * NPASS * CW
    scatter_phase(col0, stage(col0, 0), stage(col0, 1))
    plsc.subcore_barrier()

    def one_pass(p, _):
        coloff = (c * NPASS + p) * CW
        pre0 = stage(coloff, 0)
        pre1 = stage(coloff, 1)
        drain_phase(coloff - CW)
        plsc.subcore_barrier()
        scatter_phase(coloff, pre0, pre1)
        plsc.subcore_barrier()
        return _

    lax.fori_loop(1, NPASS, one_pass, None)
    drain_phase((c * NPASS + NPASS - 1) * CW)


def kernel(x, s, hashed_indices):
    s1d = s.reshape(IN_DIM)
    zeros = jnp.zeros((2 * OUT_DIM, CW), dtype=jnp.float32)
    mesh = plsc.VectorSubcoreMesh(core_axis_name="c", subcore_axis_name="s",
                                  num_cores=NC, num_subcores=NS)
    f = pl.kernel(
        _body,
        out_type=jax.ShapeDtypeStruct((OUT_DIM, COLS), jnp.float32),
        mesh=mesh,
        scratch_types=[
            pltpu.VMEM((NBLK, 128), jnp.int32),
            pltpu.VMEM((NBLK, 128), jnp.float32),
            pltpu.VMEM((CHUNK, CW), jnp.float32),
            pltpu.VMEM((CHUNK, CW), jnp.float32),
            pltpu.VMEM((OBLK, CW), jnp.float32),
            pltpu.VMEM((OBLK, CW), jnp.float32),
            pltpu.VMEM((OBLK, CW), jnp.float32),
            pltpu.VMEM((OBLK, CW), jnp.float32),
            pltpu.VMEM_SHARED((2 * OUT_DIM, CW), jnp.float32),
            pltpu.SemaphoreType.DMA,
            pltpu.SemaphoreType.DMA,
            pltpu.SemaphoreType.DMA,
            pltpu.SemaphoreType.DMA,
            pltpu.SemaphoreType.DMA,
            pltpu.SemaphoreType.DMA,
            pltpu.SemaphoreType.DMA,
            pltpu.SemaphoreType.DMA,
        ],
        compiler_params=pltpu.CompilerParams(use_tc_tiling_on_sc=False),
    )
    return f(x, s1d, hashed_indices, zeros)


# speedup vs baseline: 1.0237x; 1.0237x over previous
import jax
import jax.numpy as jnp
from jax import lax
from jax.experimental import pallas as pl
from jax.experimental.pallas import tpu as pltpu
from jax.experimental.pallas import tpu_sc as plsc

IN_DIM = 65536
OUT_DIM = 16384
COLS = 1024

NC = 2
NS = 16
L = 16

CW = 32
NPASS = COLS // (CW * NC)
RPT = IN_DIM // NS
NBLK = RPT // 128
CHUNK = 512
ZROWS = 128
OBLK = 128


def _body(x_hbm, s_hbm, idx_hbm, z_hbm, y_hbm,
          idx2_v, s_v, xbuf0, xbuf1, obuf0, nbuf0, obuf1, nbuf1, zbuf, acc,
          sem_z, sem_x0, sem_x1, sem_sc0, sem_sc1, sem_d0, sem_d1, sem_y):
    c = lax.axis_index("c")
    sid = lax.axis_index("s")
    rowbase = sid * RPT

    pltpu.sync_copy(z_hbm, zbuf)
    for b in range(NBLK):
        pltpu.sync_copy(idx_hbm.at[pl.ds(rowbase + b * 128, 128)],
                        idx2_v.at[b])
        pltpu.sync_copy(s_hbm.at[pl.ds(rowbase + b * 128, 128)],
                        s_v.at[b])

    for b in range(NBLK):
        for j in range(128 // L):
            iv = idx2_v[b, pl.ds(j * L, L)]
            sv = s_v[b, pl.ds(j * L, L)]
            bump = jnp.where(sv < 0.0, jnp.int32(OUT_DIM), jnp.int32(0))
            idx2_v[b, pl.ds(j * L, L)] = iv + bump

    zdescs = [
        pltpu.async_copy(
            zbuf,
            acc.at[pl.ds(sid * (2 * OUT_DIM // NS) + z * ZROWS, ZROWS)],
            sem_z)
        for z in range(2 * OUT_DIM // NS // ZROWS)
    ]
    for dsc in zdescs:
        dsc.wait()
    plsc.subcore_barrier()

    xb = (xbuf0, xbuf1)
    sx = (sem_x0, sem_x1)
    ssc = (sem_sc0, sem_sc1)
    NCH = RPT // CHUNK
    SPC = CHUNK // 128

    def stage(coloff, b2):
        par = b2 % 2
        return pltpu.async_copy(
            x_hbm.at[pl.ds(rowbase + b2 * CHUNK, CHUNK),
                     pl.ds(coloff, CW)],
            xb[par], sx[par])

    def scatter_phase(coloff, pre0, pre1):
        xdescs = {0: pre0, 1: pre1}
        scat_descs = {0: [], 1: []}
        for b2 in range(NCH):
            par = b2 % 2
            xdescs.pop(b2).wait()
            if 1 <= b2 and b2 + 1 < NCH:
                npar = (b2 + 1) % 2
                for dsc in scat_descs[npar]:
                    dsc.wait()
                scat_descs[npar] = []
                xdescs[b2 + 1] = stage(coloff, b2 + 1)
            for b3 in range(SPC):
                scat_descs[par].append(pltpu.async_copy(
                    xb[par].at[pl.ds(b3 * 128, 128)],
                    acc.at[idx2_v.at[b2 * SPC + b3]],
                    ssc[par], add=True))
        for par in (0, 1):
            for dsc in scat_descs[par]:
                dsc.wait()

    def drain_phase(coloff):
        orows = OUT_DIM // NS
        bufs = ((obuf0, nbuf0), (obuf1, nbuf1))
        NOB = orows // OBLK
        rdescs = {}
        ydescs = {}
        zdescs = []

        def issue_read(ob):
            o_, n_ = bufs[ob % 2]
            obase = sid * orows + ob * OBLK
            rdescs[ob] = (
                pltpu.async_copy(acc.at[pl.ds(obase, OBLK)], o_, sem_d0),
                pltpu.async_copy(acc.at[pl.ds(OUT_DIM + obase, OBLK)],
                                 n_, sem_d1))

        issue_read(0)
        for ob in range(NOB):
            o_, n_ = bufs[ob % 2]
            for dsc in rdescs.pop(ob):
                dsc.wait()
            obase = sid * orows + ob * OBLK
            zdescs.append(pltpu.async_copy(
                zbuf, acc.at[pl.ds(obase, OBLK)], sem_z))
            zdescs.append(pltpu.async_copy(
                zbuf, acc.at[pl.ds(OUT_DIM + obase, OBLK)], sem_z))
            if ob + 1 < NOB:
                if ob - 1 >= 0:
                    ydescs.pop(ob - 1).wait()
                issue_read(ob + 1)

            def sub_rows(r4, _):
                for dr in range(4):
                    r = r4 * 4 + dr
                    for j in range(CW // L):
                        o_[r, pl.ds(j * L, L)] = (
                            o_[r, pl.ds(j * L, L)]
                            - n_[r, pl.ds(j * L, L)])
                return _
            lax.fori_loop(0, OBLK // 4, sub_rows, None)

            ydescs[ob] = pltpu.async_copy(
                o_,
                y_hbm.at[pl.ds(sid * orows + ob * OBLK, OBLK),
                         pl.ds(coloff, CW)],
                sem_y)
        for dsc in ydescs.values():
            dsc.wait()
        for dsc in zdescs:
            dsc.wait()

    col0 = c * NPASS * CW
    scatter_phase(col0, stage(col0, 0), stage(col0, 1))
    plsc.subcore_barrier()

    def one_pass(p, _):
        coloff = (c * NPASS + p) * CW
        pre0 = stage(coloff, 0)
        pre1 = stage(coloff, 1)
        drain_phase(coloff - CW)
        plsc.subcore_barrier()
        scatter_phase(coloff, pre0, pre1)
        plsc.subcore_barrier()
        return _

    lax.fori_loop(1, NPASS, one_pass, None)
    drain_phase((c * NPASS + NPASS - 1) * CW)


def kernel(x, s, hashed_indices):
    s1d = s.reshape(IN_DIM)
    zeros = jnp.zeros((ZROWS, CW), dtype=jnp.float32)
    mesh = plsc.VectorSubcoreMesh(core_axis_name="c", subcore_axis_name="s",
                                  num_cores=NC, num_subcores=NS)
    f = pl.kernel(
        _body,
        out_type=jax.ShapeDtypeStruct((OUT_DIM, COLS), jnp.float32),
        mesh=mesh,
        scratch_types=[
            pltpu.VMEM((NBLK, 128), jnp.int32),
            pltpu.VMEM((NBLK, 128), jnp.float32),
            pltpu.VMEM((CHUNK, CW), jnp.float32),
            pltpu.VMEM((CHUNK, CW), jnp.float32),
            pltpu.VMEM((OBLK, CW), jnp.float32),
            pltpu.VMEM((OBLK, CW), jnp.float32),
            pltpu.VMEM((OBLK, CW), jnp.float32),
            pltpu.VMEM((OBLK, CW), jnp.float32),
            pltpu.VMEM((ZROWS, CW), jnp.float32),
            pltpu.VMEM_SHARED((2 * OUT_DIM, CW), jnp.float32),
            pltpu.SemaphoreType.DMA,
            pltpu.SemaphoreType.DMA,
            pltpu.SemaphoreType.DMA,
            pltpu.SemaphoreType.DMA,
            pltpu.SemaphoreType.DMA,
            pltpu.SemaphoreType.DMA,
            pltpu.SemaphoreType.DMA,
            pltpu.SemaphoreType.DMA,
        ],
        compiler_params=pltpu.CompilerParams(use_tc_tiling_on_sc=False),
    )
    return f(x, s1d, hashed_indices, zeros)
